# Initial kernel scaffold; baseline (speedup 1.0000x reference)
import jax
import jax.numpy as jnp
from jax.experimental import pallas as pl


def kernel(hidden_states, attention_mask, router_w, w1, b1, w2, b2):
    raise NotImplementedError("write your pallas kernel here")


# speedup vs baseline: 1.1972x; 1.1972x over previous
import functools

import jax
import jax.numpy as jnp
from jax.experimental import pallas as pl
from jax.experimental.pallas import tpu as pltpu

B, S, D, E, F, K = 1, 2048, 768, 8, 3072, 2
T = B * S
FB = 768
NF = F // FB


def _router_body(x_ref, rw_ref, mask_ref, c_ref):
    x = x_ref[...]
    logits = jnp.dot(x, rw_ref[...], preferred_element_type=jnp.float32)
    m = jnp.max(logits, axis=-1, keepdims=True)
    p = jnp.exp(logits - m)
    probs = p / jnp.sum(p, axis=-1, keepdims=True)

    lane = jax.lax.broadcasted_iota(jnp.int32, (T, E), 1)
    big = jnp.int32(E)

    v1 = jnp.max(probs, axis=-1, keepdims=True)
    i1 = jnp.min(jnp.where(probs == v1, lane, big), axis=-1, keepdims=True)
    h1 = lane == i1
    probs2 = jnp.where(h1, -jnp.inf, probs)
    v2 = jnp.max(probs2, axis=-1, keepdims=True)
    i2 = jnp.min(jnp.where(probs2 == v2, lane, big), axis=-1, keepdims=True)
    h2 = lane == i2

    denom = v1 + v2
    c = jnp.where(h1, v1 / denom, 0.0) + jnp.where(h2, v2 / denom, 0.0)
    c = c * mask_ref[...].astype(jnp.float32)
    c_ref[...] = c


def _ffn_body(x_ref, w1_ref, b1_ref, w2_ref, b2_ref, c_ref, out_ref):
    e = pl.program_id(0)
    f = pl.program_id(1)

    @pl.when(jnp.logical_and(e == 0, f == 0))
    def _():
        out_ref[...] = jnp.zeros_like(out_ref)

    x = x_ref[...]
    h = jnp.dot(x, w1_ref[0], preferred_element_type=jnp.float32)
    h = jax.nn.gelu(h + b1_ref[0])
    y = jnp.dot(h.astype(jnp.bfloat16), w2_ref[0],
                preferred_element_type=jnp.float32)
    lane = jax.lax.broadcasted_iota(jnp.int32, (T, E), 1)
    c = jnp.sum(jnp.where(lane == e, c_ref[...], 0.0), axis=1, keepdims=True)
    acc = c * y

    @pl.when(f == NF - 1)
    def _():
        out_ref[...] += acc + c * b2_ref[0]

    @pl.when(f != NF - 1)
    def _():
        out_ref[...] += acc


@jax.jit
def kernel(hidden_states, attention_mask, router_w, w1, b1, w2, b2):
    x = hidden_states.reshape(T, D)
    mask = attention_mask.reshape(T, 1)

    c = pl.pallas_call(
        _router_body,
        out_shape=jax.ShapeDtypeStruct((T, E), jnp.float32),
    )(x, router_w, mask)

    b1r = b1.reshape(E, 1, F)
    b2r = b2.reshape(E, 1, D)
    xb = x.astype(jnp.bfloat16)
    w1b = w1.astype(jnp.bfloat16)
    w2b = w2.astype(jnp.bfloat16)

    out = pl.pallas_call(
        _ffn_body,
        grid=(E, NF),
        in_specs=[
            pl.BlockSpec((T, D), lambda e, f: (0, 0)),
            pl.BlockSpec((1, D, FB), lambda e, f: (e, 0, f)),
            pl.BlockSpec((1, 1, FB), lambda e, f: (e, 0, f)),
            pl.BlockSpec((1, FB, D), lambda e, f: (e, f, 0)),
            pl.BlockSpec((1, 1, D), lambda e, f: (e, 0, 0)),
            pl.BlockSpec((T, E), lambda e, f: (0, 0)),
        ],
        out_specs=pl.BlockSpec((T, D), lambda e, f: (0, 0)),
        out_shape=jax.ShapeDtypeStruct((T, D), jnp.float32),
        compiler_params=pltpu.CompilerParams(
            dimension_semantics=("arbitrary", "arbitrary"),
        ),
    )(xb, w1b, b1r, w2b, b2r, c)

    return out.reshape(B, S, D)


# speedup vs baseline: 1.7873x; 1.4930x over previous
import functools

import jax
import jax.numpy as jnp
from jax.experimental import pallas as pl
from jax.experimental.pallas import tpu as pltpu

B, S, D, E, F, K = 1, 2048, 768, 8, 3072, 2
T = B * S
P = K * T
R = 256
NT = P // R
NS = NT + E - 1


def _dispatch_body(x_ref, rw_ref, mask_ref, xs_ref, spi_ref, spw_ref,
                   cnt_ref, r0_ref, r1_ref):
    x = x_ref[...]
    logits = jnp.dot(x, rw_ref[...], preferred_element_type=jnp.float32)
    m = jnp.max(logits, axis=-1, keepdims=True)
    p = jnp.exp(logits - m)
    probs = p / jnp.sum(p, axis=-1, keepdims=True)

    lane = jax.lax.broadcasted_iota(jnp.int32, (T, E), 1)
    big = jnp.int32(E)
    v1 = jnp.max(probs, axis=-1, keepdims=True)
    i1 = jnp.min(jnp.where(probs == v1, lane, big), axis=-1, keepdims=True)
    h1 = lane == i1
    probs2 = jnp.where(h1, -jnp.inf, probs)
    v2 = jnp.max(probs2, axis=-1, keepdims=True)
    i2 = jnp.min(jnp.where(probs2 == v2, lane, big), axis=-1, keepdims=True)
    h2 = lane == i2

    maskf = mask_ref[...].astype(jnp.float32)
    denom = v1 + v2
    w0 = v1 / denom * maskf
    w1n = v2 / denom * maskf

    h1f = h1.astype(jnp.float32)
    h2f = h2.astype(jnp.float32)
    cnt0 = jnp.sum(h1f, axis=0, keepdims=True)
    cnt1 = jnp.sum(h2f, axis=0, keepdims=True)
    counts = cnt0 + cnt1
    inc = counts
    for k in (1, 2, 4):
        inc = inc + jnp.concatenate(
            [jnp.zeros((1, k), jnp.float32), inc[:, :-k]], axis=1)
    offsets = inc - counts

    h1b = h1f.astype(jnp.bfloat16)
    h2b = h2f.astype(jnp.bfloat16)
    for it in range(T // R):
        colt = jax.lax.broadcasted_iota(jnp.int32, (R, T), 1)
        rowt = jax.lax.broadcasted_iota(jnp.int32, (R, T), 0) + it * R
        ltt = (colt < rowt).astype(jnp.bfloat16)
        r0_ref[it * R:(it + 1) * R, :] = jnp.dot(
            ltt, h1b, preferred_element_type=jnp.float32)
        r1_ref[it * R:(it + 1) * R, :] = jnp.dot(
            ltt, h2b, preferred_element_type=jnp.float32)

    pos0 = jnp.sum(h1f * (offsets + r0_ref[...]), axis=1, keepdims=True)
    pos1 = jnp.sum(h2f * (offsets + cnt0 + r1_ref[...]), axis=1,
                   keepdims=True)
    sp0 = pos0.astype(jnp.int32)
    sp1 = pos1.astype(jnp.int32)

    spi_ref[...] = jnp.concatenate([sp0, sp1], axis=1)
    spw_ref[...] = jnp.concatenate([w0, w1n], axis=1)
    cnt_ref[...] = counts.astype(jnp.int32)

    xb = x.astype(jnp.bfloat16)
    dn = (((0,), (0,)), ((), ()))
    for it in range(NT):
        piota = jax.lax.broadcasted_iota(jnp.int32, (T, R), 1) + it * R
        gt = ((sp0 == piota) | (sp1 == piota)).astype(jnp.bfloat16)
        xs = jax.lax.dot_general(gt, xb, dn,
                                 preferred_element_type=jnp.float32)
        xs_ref[it * R:(it + 1) * R, :] = xs.astype(jnp.bfloat16)


def _ffn_body(meta_ref, xs_ref, w1_ref, b1_ref, w2_ref, b2_ref, ys_ref):
    s = pl.program_id(0)
    r = meta_ref[1, s]
    start = meta_ref[2, s]
    end = meta_ref[3, s]

    xs = xs_ref[...]
    h = jnp.dot(xs, w1_ref[0], preferred_element_type=jnp.float32)
    h = jax.nn.gelu(h + b1_ref[0])
    y = jnp.dot(h.astype(jnp.bfloat16), w2_ref[0],
                preferred_element_type=jnp.float32) + b2_ref[0]

    g = jax.lax.broadcasted_iota(jnp.int32, (R, 1), 0) + r * R
    m = jnp.logical_and(g >= start, g < end)
    ys_ref[...] = jnp.where(m, y.astype(jnp.bfloat16), ys_ref[...])


def _combine_body(spi_ref, spw_ref, ys_ref, out_ref):
    sp0 = spi_ref[:, 0:1]
    sp1 = spi_ref[:, 1:2]
    w0 = spw_ref[:, 0:1]
    w1n = spw_ref[:, 1:2]
    out_ref[...] = jnp.zeros_like(out_ref)
    for it in range(NT):
        piota = jax.lax.broadcasted_iota(jnp.int32, (T, R), 1) + it * R
        h = (jnp.where(sp0 == piota, w0, 0.0)
             + jnp.where(sp1 == piota, w1n, 0.0))
        out_ref[...] += jnp.dot(h.astype(jnp.bfloat16),
                                ys_ref[it * R:(it + 1) * R, :],
                                preferred_element_type=jnp.float32)


@jax.jit
def kernel(hidden_states, attention_mask, router_w, w1, b1, w2, b2):
    x = hidden_states.reshape(T, D)
    mask = attention_mask.reshape(T, 1)

    xs, spi, spw, cnt = pl.pallas_call(
        _dispatch_body,
        out_shape=(
            jax.ShapeDtypeStruct((P, D), jnp.bfloat16),
            jax.ShapeDtypeStruct((T, K), jnp.int32),
            jax.ShapeDtypeStruct((T, K), jnp.float32),
            jax.ShapeDtypeStruct((1, E), jnp.int32),
        ),
        scratch_shapes=[
            pltpu.VMEM((T, E), jnp.float32),
            pltpu.VMEM((T, E), jnp.float32),
        ],
    )(x, router_w, mask)

    c = cnt[0]
    ends = jnp.cumsum(c)
    starts = ends - c
    tile_first = starts // R
    tile_last = (ends - 1) // R
    ntiles = jnp.where(c > 0, tile_last - tile_first + 1, 0)
    cum_nt = jnp.cumsum(ntiles)
    cum_excl = cum_nt - ntiles
    num_real = cum_nt[-1]
    s = jnp.arange(NS)
    e_of_s = jnp.sum(s[:, None] >= cum_nt[None, :], axis=1)
    valid = s < num_real
    e_s = jnp.clip(e_of_s, 0, E - 1)
    j = s - cum_excl[e_s]
    r_s = tile_first[e_s] + j
    start_s = jnp.maximum(starts[e_s], r_s * R)
    end_s = jnp.minimum(ends[e_s], (r_s + 1) * R)
    e_last = jnp.max(jnp.where(valid, e_s, 0))
    meta = jnp.stack([
        jnp.where(valid, e_s, e_last),
        jnp.where(valid, r_s, NT - 1),
        jnp.where(valid, start_s, 0),
        jnp.where(valid, end_s, 0),
    ]).astype(jnp.int32)

    w1b = w1.astype(jnp.bfloat16)
    w2b = w2.astype(jnp.bfloat16)
    b1r = b1.reshape(E, 1, F)
    b2r = b2.reshape(E, 1, D)

    ys = pl.pallas_call(
        _ffn_body,
        grid_spec=pltpu.PrefetchScalarGridSpec(
            num_scalar_prefetch=1,
            grid=(NS,),
            in_specs=[
                pl.BlockSpec((R, D), lambda s, m: (m[1, s], 0)),
                pl.BlockSpec((1, D, F), lambda s, m: (m[0, s], 0, 0)),
                pl.BlockSpec((1, 1, F), lambda s, m: (m[0, s], 0, 0)),
                pl.BlockSpec((1, F, D), lambda s, m: (m[0, s], 0, 0)),
                pl.BlockSpec((1, 1, D), lambda s, m: (m[0, s], 0, 0)),
            ],
            out_specs=pl.BlockSpec((R, D), lambda s, m: (m[1, s], 0)),
        ),
        out_shape=jax.ShapeDtypeStruct((P, D), jnp.bfloat16),
        compiler_params=pltpu.CompilerParams(
            dimension_semantics=("arbitrary",),
        ),
    )(meta, xs, w1b, b1r, w2b, b2r)

    out = pl.pallas_call(
        _combine_body,
        out_shape=jax.ShapeDtypeStruct((T, D), jnp.float32),
    )(spi, spw, ys)

    return out.reshape(B, S, D)


# speedup vs baseline: 2.1411x; 1.1980x over previous
import functools

import jax
import jax.numpy as jnp
from jax import lax
from jax.experimental import pallas as pl
from jax.experimental.pallas import tpu as pltpu
from jax.experimental.pallas import tpu_sc as plsc

B, S, D, E, F, K = 1, 2048, 768, 8, 3072, 2
T = B * S
P = K * T
R = 256
NT = P // R
F4 = F // 4


def _mega_body(x_ref, rw_ref, mask_ref, w1_ref, b1_ref, w2_ref, b2_ref,
               ys_ref, spi_ref, spw0_ref, spw1_ref,
               xs_ref, offs_ref, r0_ref, r1_ref, w1c_ref, w2c_ref):
    eg = pl.program_id(0)
    fh = pl.program_id(1)

    @pl.when(jnp.logical_and(eg == 0, fh == 0))
    def _dispatch():
        x = x_ref[...]
        logits = jnp.dot(x, rw_ref[...], preferred_element_type=jnp.float32)
        m = jnp.max(logits, axis=-1, keepdims=True)
        p = jnp.exp(logits - m)
        probs = p / jnp.sum(p, axis=-1, keepdims=True)

        lane = jax.lax.broadcasted_iota(jnp.int32, (T, E), 1)
        big = jnp.int32(E)
        v1 = jnp.max(probs, axis=-1, keepdims=True)
        i1 = jnp.min(jnp.where(probs == v1, lane, big), axis=-1,
                     keepdims=True)
        h1 = lane == i1
        probs2 = jnp.where(h1, -jnp.inf, probs)
        v2 = jnp.max(probs2, axis=-1, keepdims=True)
        i2 = jnp.min(jnp.where(probs2 == v2, lane, big), axis=-1,
                     keepdims=True)
        h2 = lane == i2

        maskf = mask_ref[...].astype(jnp.float32)
        denom = v1 + v2
        w0 = v1 / denom * maskf
        w1n = v2 / denom * maskf

        h1f = h1.astype(jnp.float32)
        h2f = h2.astype(jnp.float32)
        cnt0 = jnp.sum(h1f, axis=0, keepdims=True)
        cnt1 = jnp.sum(h2f, axis=0, keepdims=True)
        counts = cnt0 + cnt1
        inc = counts
        for k in (1, 2, 4):
            inc = inc + jnp.concatenate(
                [jnp.zeros((1, k), jnp.float32), inc[:, :-k]], axis=1)
        offsets = inc - counts
        offs_ref[...] = jnp.concatenate([offsets, inc], axis=1)

        h1b = h1f.astype(jnp.bfloat16)
        h2b = h2f.astype(jnp.bfloat16)
        for it in range(T // R):
            colt = jax.lax.broadcasted_iota(jnp.int32, (R, T), 1)
            rowt = jax.lax.broadcasted_iota(jnp.int32, (R, T), 0) + it * R
            ltt = (colt < rowt).astype(jnp.bfloat16)
            r0_ref[it * R:(it + 1) * R, :] = jnp.dot(
                ltt, h1b, preferred_element_type=jnp.float32)
            r1_ref[it * R:(it + 1) * R, :] = jnp.dot(
                ltt, h2b, preferred_element_type=jnp.float32)

        pos0 = jnp.sum(h1f * (offsets + r0_ref[...]), axis=1, keepdims=True)
        pos1 = jnp.sum(h2f * (offsets + cnt0 + r1_ref[...]), axis=1,
                       keepdims=True)
        sp0 = pos0.astype(jnp.int32)
        sp1 = pos1.astype(jnp.int32)

        spi_ref[...] = jnp.concatenate([sp0, sp1], axis=1)
        spw0_ref[...] = jnp.broadcast_to(w0, (T, 16))
        spw1_ref[...] = jnp.broadcast_to(w1n, (T, 16))

        xb = x.astype(jnp.bfloat16)
        dn = (((0,), (0,)), ((), ()))
        for it in range(NT):
            piota = jax.lax.broadcasted_iota(jnp.int32, (T, R), 1) + it * R
            gt = ((sp0 == piota) | (sp1 == piota)).astype(jnp.bfloat16)
            xs = jax.lax.dot_general(gt, xb, dn,
                                     preferred_element_type=jnp.float32)
            xs_ref[it * R:(it + 1) * R, :] = xs.astype(jnp.bfloat16)

    @pl.when(eg > 0)
    def _ffn():
        e = eg - 1
        vec = offs_ref[...]
        lane = jax.lax.broadcasted_iota(jnp.int32, (1, 2 * E), 1)
        start = jnp.sum(jnp.where(lane == e, vec, 0.0)).astype(jnp.int32)
        end = jnp.sum(jnp.where(lane == e + E, vec, 0.0)).astype(jnp.int32)
        tf = lax.div(start, R)
        nt = jnp.where(end > start, lax.div(end - 1, R) - tf + 1, 0)

        w1c_ref[...] = w1_ref[0].astype(jnp.bfloat16)
        w2c_ref[...] = w2_ref[0].astype(jnp.bfloat16)
        fh0 = (fh == 0)

        def chunk(i, carry):
            base = (tf + i) * R
            xsc = xs_ref[pl.ds(base, R), :]
            h = jnp.dot(xsc, w1c_ref[...],
                        preferred_element_type=jnp.float32)
            h = jax.nn.gelu(h + b1_ref[0])
            y = jnp.dot(h.astype(jnp.bfloat16), w2c_ref[...],
                        preferred_element_type=jnp.float32)
            g = base + jax.lax.broadcasted_iota(jnp.int32, (R, 1), 0)
            msk = jnp.logical_and(g >= start, g < end)
            old = ys_ref[pl.ds(base, R), :]
            acc = jnp.where(fh0, 0.0, old) + y \
                + jnp.where(fh0, 1.0, 0.0) * b2_ref[0]
            ys_ref[pl.ds(base, R), :] = jnp.where(msk, acc, old)
            return carry

        jax.lax.fori_loop(0, nt, chunk, 0)


NC, NSUB = 2, 16
NWK = NC * NSUB
TPW = T // NWK
_SC_MESH = plsc.VectorSubcoreMesh(core_axis_name="c", subcore_axis_name="s")


@functools.partial(
    pl.kernel,
    mesh=_SC_MESH,
    out_type=jax.ShapeDtypeStruct((T, D), jnp.float32),
    scratch_types=[
        pltpu.VMEM((TPW,), jnp.int32),
        pltpu.VMEM((TPW,), jnp.int32),
        pltpu.VMEM((TPW, 16), jnp.float32),
        pltpu.VMEM((TPW, 16), jnp.float32),
        pltpu.VMEM((TPW, D), jnp.float32),
        pltpu.VMEM((TPW, D), jnp.float32),
        pltpu.SemaphoreType.DMA,
    ],
)
def _sc_combine(ys_hbm, sp0_hbm, sp1_hbm, w0_hbm, w1_hbm, out_hbm,
                idx0_v, idx1_v, w0_v, w1_v, r0_v, r1_v, sem):
    wid = lax.axis_index("s") * NC + lax.axis_index("c")
    base = wid * TPW
    pltpu.sync_copy(sp0_hbm.at[pl.ds(base, TPW)], idx0_v)
    pltpu.sync_copy(sp1_hbm.at[pl.ds(base, TPW)], idx1_v)
    pltpu.sync_copy(w0_hbm.at[pl.ds(base, TPW)], w0_v)
    pltpu.sync_copy(w1_hbm.at[pl.ds(base, TPW)], w1_v)
    c0 = pltpu.async_copy(ys_hbm.at[idx0_v], r0_v, sem)
    c1 = pltpu.async_copy(ys_hbm.at[idx1_v], r1_v, sem)
    c0.wait()
    c1.wait()

    def tok(t, carry):
        s0 = w0_v[t, pl.ds(0, 16)]
        s1 = w1_v[t, pl.ds(0, 16)]
        for j in range(D // 16):
            a = r0_v[t, pl.ds(j * 16, 16)]
            b = r1_v[t, pl.ds(j * 16, 16)]
            r0_v[t, pl.ds(j * 16, 16)] = a * s0 + b * s1
        return carry

    jax.lax.fori_loop(0, TPW, tok, 0)
    pltpu.sync_copy(r0_v, out_hbm.at[pl.ds(base, TPW)])


@jax.jit
def kernel(hidden_states, attention_mask, router_w, w1, b1, w2, b2):
    x = hidden_states.reshape(T, D)
    mask = attention_mask.reshape(T, 1)
    b1r = b1.reshape(E, 1, F)
    b2r = b2.reshape(E, 1, D)

    ys, spi, spw0, spw1 = pl.pallas_call(
        _mega_body,
        grid=(E + 1, 4),
        in_specs=[
            pl.BlockSpec((T, D), lambda e, f: (0, 0)),
            pl.BlockSpec((D, E), lambda e, f: (0, 0)),
            pl.BlockSpec((T, 1), lambda e, f: (0, 0)),
            pl.BlockSpec((1, D, F4),
                         lambda e, f: (jnp.maximum(e - 1, 0), 0,
                                       jnp.where(e == 0, 0, f))),
            pl.BlockSpec((1, 1, F4),
                         lambda e, f: (jnp.maximum(e - 1, 0), 0,
                                       jnp.where(e == 0, 0, f))),
            pl.BlockSpec((1, F4, D),
                         lambda e, f: (jnp.maximum(e - 1, 0),
                                       jnp.where(e == 0, 0, f), 0)),
            pl.BlockSpec((1, 1, D),
                         lambda e, f: (jnp.maximum(e - 1, 0), 0, 0)),
        ],
        out_specs=(
            pl.BlockSpec((P, D), lambda e, f: (0, 0)),
            pl.BlockSpec((T, K), lambda e, f: (0, 0)),
            pl.BlockSpec((T, 16), lambda e, f: (0, 0)),
            pl.BlockSpec((T, 16), lambda e, f: (0, 0)),
        ),
        out_shape=(
            jax.ShapeDtypeStruct((P, D), jnp.float32),
            jax.ShapeDtypeStruct((T, K), jnp.int32),
            jax.ShapeDtypeStruct((T, 16), jnp.float32),
            jax.ShapeDtypeStruct((T, 16), jnp.float32),
        ),
        scratch_shapes=[
            pltpu.VMEM((P, D), jnp.bfloat16),
            pltpu.VMEM((1, 2 * E), jnp.float32),
            pltpu.VMEM((T, E), jnp.float32),
            pltpu.VMEM((T, E), jnp.float32),
            pltpu.VMEM((D, F4), jnp.bfloat16),
            pltpu.VMEM((F4, D), jnp.bfloat16),
        ],
        compiler_params=pltpu.CompilerParams(
            dimension_semantics=("arbitrary", "arbitrary"),
            vmem_limit_bytes=100 * 1024 * 1024,
        ),
    )(x, router_w, mask, w1, b1r, w2, b2r)

    out = _sc_combine(ys, spi[:, 0], spi[:, 1], spw0, spw1)
    return out.reshape(B, S, D)


# speedup vs baseline: 2.2528x; 1.0522x over previous
import functools

import jax
import jax.numpy as jnp
from jax import lax
from jax.experimental import pallas as pl
from jax.experimental.pallas import tpu as pltpu
from jax.experimental.pallas import tpu_sc as plsc

B, S, D, E, F, K = 1, 2048, 768, 8, 3072, 2
T = B * S
P = K * T
R = 256
NT = P // R
NS = NT + E - 1


def _dispatch_body(x_ref, rw_ref, mask_ref, xs_ref, spi_ref, spw_ref,
                   cnt_ref, r0_ref, r1_ref):
    x = x_ref[...]
    logits = jnp.dot(x, rw_ref[...], preferred_element_type=jnp.float32)
    m = jnp.max(logits, axis=-1, keepdims=True)
    p = jnp.exp(logits - m)
    probs = p / jnp.sum(p, axis=-1, keepdims=True)

    lane = jax.lax.broadcasted_iota(jnp.int32, (T, E), 1)
    big = jnp.int32(E)
    v1 = jnp.max(probs, axis=-1, keepdims=True)
    i1 = jnp.min(jnp.where(probs == v1, lane, big), axis=-1, keepdims=True)
    h1 = lane == i1
    probs2 = jnp.where(h1, -jnp.inf, probs)
    v2 = jnp.max(probs2, axis=-1, keepdims=True)
    i2 = jnp.min(jnp.where(probs2 == v2, lane, big), axis=-1, keepdims=True)
    h2 = lane == i2

    maskf = mask_ref[...].astype(jnp.float32)
    denom = v1 + v2
    w0 = v1 / denom * maskf
    w1n = v2 / denom * maskf

    h1f = h1.astype(jnp.float32)
    h2f = h2.astype(jnp.float32)
    cnt0 = jnp.sum(h1f, axis=0, keepdims=True)
    cnt1 = jnp.sum(h2f, axis=0, keepdims=True)
    counts = cnt0 + cnt1
    inc = counts
    for k in (1, 2, 4):
        inc = inc + jnp.concatenate(
            [jnp.zeros((1, k), jnp.float32), inc[:, :-k]], axis=1)
    offsets = inc - counts

    h1b = h1f.astype(jnp.bfloat16)
    h2b = h2f.astype(jnp.bfloat16)
    for it in range(T // R):
        colt = jax.lax.broadcasted_iota(jnp.int32, (R, T), 1)
        rowt = jax.lax.broadcasted_iota(jnp.int32, (R, T), 0) + it * R
        ltt = (colt < rowt).astype(jnp.bfloat16)
        r0_ref[it * R:(it + 1) * R, :] = jnp.dot(
            ltt, h1b, preferred_element_type=jnp.float32)
        r1_ref[it * R:(it + 1) * R, :] = jnp.dot(
            ltt, h2b, preferred_element_type=jnp.float32)

    pos0 = jnp.sum(h1f * (offsets + r0_ref[...]), axis=1, keepdims=True)
    pos1 = jnp.sum(h2f * (offsets + cnt0 + r1_ref[...]), axis=1,
                   keepdims=True)
    sp0 = pos0.astype(jnp.int32)
    sp1 = pos1.astype(jnp.int32)

    spi_ref[...] = jnp.concatenate([sp0, sp1], axis=1)
    spw_ref[...] = jnp.concatenate([w0, w1n], axis=1)
    cnt_ref[...] = counts.astype(jnp.int32)

    xb = x.astype(jnp.bfloat16)
    dn = (((0,), (0,)), ((), ()))
    for it in range(NT):
        piota = jax.lax.broadcasted_iota(jnp.int32, (T, R), 1) + it * R
        gt = ((sp0 == piota) | (sp1 == piota)).astype(jnp.bfloat16)
        xs = jax.lax.dot_general(gt, xb, dn,
                                 preferred_element_type=jnp.float32)
        xs_ref[it * R:(it + 1) * R, :] = xs.astype(jnp.bfloat16)


F2 = F // 2


def _ffn_body(meta_ref, xs_ref, w1_ref, b1_ref, w2_ref, b2_ref, ys_ref,
              w1c_ref, w2c_ref):
    e = pl.program_id(0)
    fh = pl.program_id(1)
    tf = meta_ref[0, e]
    nt = meta_ref[1, e]
    start = meta_ref[2, e]
    end = meta_ref[3, e]

    w1c_ref[...] = w1_ref[0].astype(jnp.bfloat16)
    w2c_ref[...] = w2_ref[0].astype(jnp.bfloat16)
    fh0 = (fh == 0)

    def chunk(i, carry):
        base = (tf + i) * R
        xs = xs_ref[pl.ds(base, R), :]
        h = jnp.dot(xs, w1c_ref[...], preferred_element_type=jnp.float32)
        h = jax.nn.gelu(h + b1_ref[0])
        y = jnp.dot(h.astype(jnp.bfloat16), w2c_ref[...],
                    preferred_element_type=jnp.float32)
        g = base + jax.lax.broadcasted_iota(jnp.int32, (R, 1), 0)
        m = jnp.logical_and(g >= start, g < end)
        old = ys_ref[pl.ds(base, R), :]
        acc = jnp.where(fh0, 0.0, old) + y \
            + jnp.where(fh0, 1.0, 0.0) * b2_ref[0]
        ys_ref[pl.ds(base, R), :] = jnp.where(m, acc, old)
        return carry

    jax.lax.fori_loop(0, nt, chunk, 0)


NC, NSUB = 2, 16
NWK = NC * NSUB
TPW = T // NWK
_SC_MESH = plsc.VectorSubcoreMesh(core_axis_name="c", subcore_axis_name="s")


@functools.partial(
    pl.kernel,
    mesh=_SC_MESH,
    out_type=jax.ShapeDtypeStruct((T, D), jnp.float32),
    scratch_types=[
        pltpu.VMEM((TPW,), jnp.int32),
        pltpu.VMEM((TPW,), jnp.int32),
        pltpu.VMEM((TPW, 16), jnp.float32),
        pltpu.VMEM((TPW, 16), jnp.float32),
        pltpu.VMEM((TPW, D), jnp.float32),
        pltpu.VMEM((TPW, D), jnp.float32),
        pltpu.SemaphoreType.DMA,
    ],
)
def _sc_combine(ys_hbm, sp0_hbm, sp1_hbm, w0_hbm, w1_hbm, out_hbm,
                idx0_v, idx1_v, w0_v, w1_v, r0_v, r1_v, sem):
    wid = lax.axis_index("s") * NC + lax.axis_index("c")
    base = wid * TPW
    pltpu.sync_copy(sp0_hbm.at[pl.ds(base, TPW)], idx0_v)
    pltpu.sync_copy(sp1_hbm.at[pl.ds(base, TPW)], idx1_v)
    pltpu.sync_copy(w0_hbm.at[pl.ds(base, TPW)], w0_v)
    pltpu.sync_copy(w1_hbm.at[pl.ds(base, TPW)], w1_v)
    c0 = pltpu.async_copy(ys_hbm.at[idx0_v], r0_v, sem)
    c1 = pltpu.async_copy(ys_hbm.at[idx1_v], r1_v, sem)
    c0.wait()
    c1.wait()

    def tok(t, carry):
        s0 = w0_v[t, pl.ds(0, 16)]
        s1 = w1_v[t, pl.ds(0, 16)]
        for j in range(D // 16):
            a = r0_v[t, pl.ds(j * 16, 16)]
            b = r1_v[t, pl.ds(j * 16, 16)]
            r0_v[t, pl.ds(j * 16, 16)] = a * s0 + b * s1
        return carry

    jax.lax.fori_loop(0, TPW, tok, 0)
    pltpu.sync_copy(r0_v, out_hbm.at[pl.ds(base, TPW)])


@jax.jit
def kernel(hidden_states, attention_mask, router_w, w1, b1, w2, b2):
    x = hidden_states.reshape(T, D)
    mask = attention_mask.reshape(T, 1)

    xs, spi, spw, cnt = pl.pallas_call(
        _dispatch_body,
        out_shape=(
            jax.ShapeDtypeStruct((P, D), jnp.bfloat16),
            jax.ShapeDtypeStruct((T, K), jnp.int32),
            jax.ShapeDtypeStruct((T, K), jnp.float32),
            jax.ShapeDtypeStruct((1, E), jnp.int32),
        ),
        scratch_shapes=[
            pltpu.VMEM((T, E), jnp.float32),
            pltpu.VMEM((T, E), jnp.float32),
        ],
    )(x, router_w, mask)

    c = cnt[0]
    ends = jnp.cumsum(c)
    starts = ends - c
    tile_first = starts // R
    tile_last = (ends - 1) // R
    ntiles = jnp.where(c > 0, tile_last - tile_first + 1, 0)
    meta = jnp.stack([tile_first, ntiles, starts, ends]).astype(jnp.int32)

    b1r = b1.reshape(E, 1, F)
    b2r = b2.reshape(E, 1, D)

    ys = pl.pallas_call(
        _ffn_body,
        grid_spec=pltpu.PrefetchScalarGridSpec(
            num_scalar_prefetch=1,
            grid=(E, 2),
            in_specs=[
                pl.BlockSpec((P, D), lambda e, f, m: (0, 0)),
                pl.BlockSpec((1, D, F2), lambda e, f, m: (e, 0, f)),
                pl.BlockSpec((1, 1, F2), lambda e, f, m: (e, 0, f)),
                pl.BlockSpec((1, F2, D), lambda e, f, m: (e, f, 0)),
                pl.BlockSpec((1, 1, D), lambda e, f, m: (e, 0, 0)),
            ],
            out_specs=pl.BlockSpec((P, D), lambda e, f, m: (0, 0)),
            scratch_shapes=[
                pltpu.VMEM((D, F2), jnp.bfloat16),
                pltpu.VMEM((F2, D), jnp.bfloat16),
            ],
        ),
        out_shape=jax.ShapeDtypeStruct((P, D), jnp.float32),
        compiler_params=pltpu.CompilerParams(
            dimension_semantics=("arbitrary", "arbitrary"),
        ),
    )(meta, xs, w1, b1r, w2, b2r)

    out = _sc_combine(
        ys,
        spi[:, 0],
        spi[:, 1],
        jnp.broadcast_to(spw[:, 0:1], (T, 16)),
        jnp.broadcast_to(spw[:, 1:2], (T, 16)),
    )

    return out.reshape(B, S, D)
